# Initial kernel scaffold; baseline (speedup 1.0000x reference)
import jax
import jax.numpy as jnp
from jax.experimental import pallas as pl


def kernel(word_repr, sent_repr, edge_index, W_src, W_dst, attn_l, attn_r, bias_gat, w1, b1, w2, b2):
    raise NotImplementedError("write your pallas kernel here")


# speedup vs baseline: 18.2418x; 18.2418x over previous
import functools

import jax
import jax.numpy as jnp
from jax import lax
from jax.experimental import pallas as pl
from jax.experimental.pallas import tpu as pltpu
from jax.experimental.pallas import tpu_sc as plsc

NWD = 10000
NST = 10000
NEDGE = 160000
DIM = 256
NH = 8
HDIM = 32
NFF = 512
NCORE = 2
NTILE = 16
NG = 4
GC = 64
CC = 128
TPG = 8
EPTA = NEDGE // TPG
CHA = 400
NCHA = EPTA // CHA
EPTB = NEDGE // NTILE
CHB = 80
NCHB = EPTB // CHB
TAB = NWD * 2
TABP = 20480
RED = TABP // TPG
ROWB = 2000


def _pre_body(sent_ref, word_ref, wsrc_ref, wdst_ref, al_ref, ar_ref,
              feat_ref, el_ref, er_ref, mx_ref):
    i = pl.program_id(0)
    fs = jnp.dot(sent_ref[...], wsrc_ref[...],
                 preferred_element_type=jnp.float32)
    fd = jnp.dot(word_ref[...], wdst_ref[...],
                 preferred_element_type=jnp.float32)
    el = jnp.sum(fs.reshape(ROWB, NH, HDIM) * al_ref[...][None], axis=-1)
    er = jnp.sum(fd.reshape(ROWB, NH, HDIM) * ar_ref[...][None], axis=-1)
    feat_ref[0] = fs[:, :CC]
    feat_ref[1] = fs[:, CC:]
    for g in range(NG):
        el_ref[g] = el[:, 2 * g:2 * g + 2]
        er_ref[g] = er[:, 2 * g:2 * g + 2]
    bm = jnp.max(el, axis=0).reshape(NG, 2)
    bm = jnp.concatenate(
        [bm, jnp.full((NG, 126), -1e30, jnp.float32)], axis=1)

    @pl.when(i == 0)
    def _():
        mx_ref[...] = bm

    @pl.when(i > 0)
    def _():
        mx_ref[...] = jnp.maximum(mx_ref[...], bm)


def _pre(sent, word, wsrc, wdst, al, ar):
    grid = NST // ROWB
    return pl.pallas_call(
        _pre_body,
        grid=(grid,),
        in_specs=[
            pl.BlockSpec((ROWB, DIM), lambda i: (i, 0)),
            pl.BlockSpec((ROWB, DIM), lambda i: (i, 0)),
            pl.BlockSpec((DIM, DIM), lambda i: (0, 0)),
            pl.BlockSpec((DIM, DIM), lambda i: (0, 0)),
            pl.BlockSpec((NH, HDIM), lambda i: (0, 0)),
            pl.BlockSpec((NH, HDIM), lambda i: (0, 0)),
        ],
        out_specs=[
            pl.BlockSpec((NCORE, ROWB, CC), lambda i: (0, i, 0)),
            pl.BlockSpec((NG, ROWB, 2), lambda i: (0, i, 0)),
            pl.BlockSpec((NG, ROWB, 2), lambda i: (0, i, 0)),
            pl.BlockSpec((NG, 128), lambda i: (0, 0)),
        ],
        out_shape=[
            jax.ShapeDtypeStruct((NCORE, NST, CC), jnp.float32),
            jax.ShapeDtypeStruct((NG, NST, 2), jnp.float32),
            jax.ShapeDtypeStruct((NG, NWD, 2), jnp.float32),
            jax.ShapeDtypeStruct((NG, 128), jnp.float32),
        ],
    )(sent, word, wsrc, wdst, al, ar)


def _lrelu(x):
    return jnp.where(x >= 0.0, x, 0.2 * x)


def _sc_edge_softmax(el4, er4, mx4, src, dst):
    mesh = plsc.VectorSubcoreMesh(core_axis_name="c", subcore_axis_name="s")

    @functools.partial(
        pl.kernel,
        out_type=jax.ShapeDtypeStruct((NG * NEDGE * 2,), jnp.float32),
        mesh=mesh,
        compiler_params=pltpu.CompilerParams(needs_layout_passes=False),
        scratch_types=[
            pltpu.VMEM((TAB,), jnp.float32),
            pltpu.VMEM((TAB,), jnp.float32),
            pltpu.VMEM((128,), jnp.float32),
            pltpu.VMEM((TABP,), jnp.float32),
            pltpu.VMEM((CHA,), jnp.int32),
            pltpu.VMEM((CHA,), jnp.int32),
            pltpu.VMEM((CHA * 2,), jnp.float32),
            pltpu.VMEM((RED,), jnp.float32),
            pltpu.VMEM((RED,), jnp.float32),
            pltpu.VMEM_SHARED((NTILE, TABP), jnp.float32),
            pltpu.VMEM_SHARED((2, TABP), jnp.float32),
        ],
    )
    def kern(el_h, er_h, mx_h, src_h, dst_h, ex_h,
             el_t, er_t, mx_t, den_t, src_b, dst_b, ex_b, tmp_b, acc_b,
             slab, dfull):
        c = lax.axis_index("c")
        s = lax.axis_index("s")
        g = s >> 3
        st = s & 7
        cg = c * 2 + g
        pltpu.sync_copy(el_h.at[cg], el_t)
        pltpu.sync_copy(er_h.at[cg], er_t)
        pltpu.sync_copy(mx_h.at[cg], mx_t)

        def zloop(v, _):
            den_t[pl.ds(v * 16, 16)] = jnp.zeros((16,), jnp.float32)
            return 0
        lax.fori_loop(0, TABP // 16, zloop, 0)

        def chunk(i, _):
            off = st * EPTA + i * CHA
            pltpu.sync_copy(src_h.at[pl.ds(pl.multiple_of(off, 8), CHA)],
                            src_b)
            pltpu.sync_copy(dst_h.at[pl.ds(pl.multiple_of(off, 8), CHA)],
                            dst_b)

            def oct_(v, _):
                iv2 = lax.iota(jnp.int32, 16) >> 1
                im2 = lax.iota(jnp.int32, 16) & 1
                mxe = plsc.load_gather(mx_t, [im2])
                eb = v * 8
                sidx = plsc.load_gather(src_b, [iv2 + eb]) * 2 + im2
                didx = plsc.load_gather(dst_b, [iv2 + eb]) * 2 + im2
                elg = plsc.load_gather(el_t, [sidx])
                erg = plsc.load_gather(er_t, [didx])
                e = _lrelu(elg + erg)
                k = _lrelu(mxe + erg)
                ex = jnp.exp(e - k)
                ex_b[pl.ds(v * 16, 16)] = ex
                for p in range(8):
                    plsc.addupdate_scatter(den_t, [didx], ex, mask=iv2 == p)
                return 0
            lax.fori_loop(0, CHA // 8, oct_, 0)
            pltpu.sync_copy(
                ex_b,
                ex_h.at[pl.ds(
                    pl.multiple_of(cg * (NEDGE * 2) + off * 2, 8), CHA * 2)])
            return 0
        lax.fori_loop(0, NCHA, chunk, 0)

        pltpu.sync_copy(den_t, slab.at[s])
        plsc.subcore_barrier()
        base = g * TPG
        roff = pl.multiple_of(st * RED, 8)
        pltpu.sync_copy(slab.at[base, pl.ds(roff, RED)], acc_b)

        def red(t, _):
            pltpu.sync_copy(slab.at[t, pl.ds(roff, RED)], tmp_b)

            def radd(v, _):
                w = pl.ds(v * 16, 16)
                acc_b[w] = acc_b[w] + tmp_b[w]
                return 0
            lax.fori_loop(0, RED // 16, radd, 0)
            return 0
        lax.fori_loop(base + 1, base + TPG, red, 0)

        def inv(v, _):
            w = pl.ds(v * 16, 16)
            acc_b[w] = 1.0 / jnp.maximum(acc_b[w], 1e-9)
            return 0
        lax.fori_loop(0, RED // 16, inv, 0)
        pltpu.sync_copy(acc_b, dfull.at[g, pl.ds(roff, RED)])
        plsc.subcore_barrier()
        pltpu.sync_copy(dfull.at[g], den_t)

        def chunk2(i, _):
            off = st * EPTA + i * CHA
            pltpu.sync_copy(dst_h.at[pl.ds(pl.multiple_of(off, 8), CHA)],
                            dst_b)
            eslice = pl.ds(
                pl.multiple_of(cg * (NEDGE * 2) + off * 2, 8), CHA * 2)
            pltpu.sync_copy(ex_h.at[eslice], ex_b)

            def oct_(v, _):
                iv2 = lax.iota(jnp.int32, 16) >> 1
                im2 = lax.iota(jnp.int32, 16) & 1
                didx = plsc.load_gather(dst_b, [iv2 + v * 8]) * 2 + im2
                dv = plsc.load_gather(den_t, [didx])
                w = pl.ds(v * 16, 16)
                ex_b[w] = ex_b[w] * dv
                return 0
            lax.fori_loop(0, CHA // 8, oct_, 0)
            pltpu.sync_copy(ex_b, ex_h.at[eslice])
            return 0
        lax.fori_loop(0, NCHA, chunk2, 0)

    return kern(el4, er4, mx4, src, dst)


def _sc_aggregate(feat2, src, dst, alphaf):
    mesh = plsc.VectorSubcoreMesh(core_axis_name="c", subcore_axis_name="s")

    @functools.partial(
        pl.kernel,
        out_type=jax.ShapeDtypeStruct((NCORE, NWD, CC), jnp.float32),
        mesh=mesh,
        compiler_params=pltpu.CompilerParams(needs_layout_passes=False),
        scratch_types=[
            pltpu.VMEM((CHB, CC), jnp.float32),
            pltpu.VMEM((CHB,), jnp.int32),
            pltpu.VMEM((CHB,), jnp.int32),
            pltpu.VMEM((CHB * 2,), jnp.float32),
            pltpu.VMEM((CHB * 2,), jnp.float32),
            pltpu.VMEM_SHARED((NWD, CC), jnp.float32),
        ],
    )
    def kern(feat_h, src_h, dst_h, a_h, rst_h,
             fbuf, src_b, dst_b, a0_b, a1_b, racc):
        c = lax.axis_index("c")
        s = lax.axis_index("s")

        def zloop(v, _):
            fbuf[v >> 3, pl.ds((v & 7) * 16, 16)] = jnp.zeros(
                (16,), jnp.float32)
            return 0
        lax.fori_loop(0, CHB * (CC // 16), zloop, 0)

        z0 = s * 640

        @pl.when(s < 15)
        def _():
            def zcopy(r, _):
                pltpu.sync_copy(
                    fbuf,
                    racc.at[pl.ds(pl.multiple_of(z0 + r * CHB, 8), CHB)])
                return 0
            lax.fori_loop(0, 640 // CHB, zcopy, 0)

        @pl.when(s == 15)
        def _():
            def zcopy(r, _):
                pltpu.sync_copy(
                    fbuf,
                    racc.at[pl.ds(pl.multiple_of(z0 + r * CHB, 8), CHB)])
                return 0
            lax.fori_loop(0, 400 // CHB, zcopy, 0)
        plsc.subcore_barrier()

        def chunk(i, _):
            off = s * EPTB + i * CHB
            pltpu.sync_copy(src_h.at[pl.ds(pl.multiple_of(off, 8), CHB)],
                            src_b)
            pltpu.sync_copy(dst_h.at[pl.ds(pl.multiple_of(off, 8), CHB)],
                            dst_b)
            pltpu.sync_copy(
                a_h.at[pl.ds(
                    pl.multiple_of((2 * c) * (NEDGE * 2) + off * 2, 8),
                    CHB * 2)], a0_b)
            pltpu.sync_copy(
                a_h.at[pl.ds(
                    pl.multiple_of((2 * c + 1) * (NEDGE * 2) + off * 2, 8),
                    CHB * 2)], a1_b)

            def soff(v, _):
                w = pl.ds(v * 16, 16)
                src_b[w] = src_b[w] + c * NST
                return 0
            lax.fori_loop(0, CHB // 16, soff, 0)
            pltpu.sync_copy(feat_h.at[src_b], fbuf)

            def scale(e, _):
                for h in range(4):
                    ab = a0_b if h < 2 else a1_b
                    av = plsc.load_gather(
                        ab, [jnp.full((16,), 0, jnp.int32) + e * 2 + (h & 1)])
                    for j in range(2):
                        w = pl.ds(h * HDIM + j * 16, 16)
                        fbuf[e, w] = fbuf[e, w] * av
                return 0
            lax.fori_loop(0, CHB, scale, 0)
            pltpu.sync_copy(fbuf, racc.at[dst_b], add=True)
            return 0
        lax.fori_loop(0, NCHB, chunk, 0)

        plsc.subcore_barrier()

        @pl.when(s < 15)
        def _():
            pltpu.sync_copy(
                racc.at[pl.ds(pl.multiple_of(z0, 8), 640)],
                rst_h.at[c, pl.ds(pl.multiple_of(z0, 8), 640)])

        @pl.when(s == 15)
        def _():
            pltpu.sync_copy(racc.at[pl.ds(9600, 400)],
                            rst_h.at[c, pl.ds(9600, 400)])

    return kern(feat2, src, dst, alphaf)


def _ffn_body(word_ref, rst_ref, bg_ref, w1_ref, b1_ref, w2_ref, b2_ref,
              out_ref):
    r = jnp.concatenate([rst_ref[0], rst_ref[1]], axis=1) + bg_ref[...][None]
    u = jnp.where(r > 0.0, r, jnp.exp(jnp.minimum(r, 0.0)) - 1.0)
    h = word_ref[...] + u
    t = jnp.maximum(
        jnp.dot(h, w1_ref[...], preferred_element_type=jnp.float32)
        + b1_ref[...][None], 0.0)
    out_ref[...] = (jnp.dot(t, w2_ref[...], preferred_element_type=jnp.float32)
                    + b2_ref[...][None])


def _ffn(word, rst, bg, w1, b1, w2, b2):
    grid = NWD // ROWB
    return pl.pallas_call(
        _ffn_body,
        grid=(grid,),
        in_specs=[
            pl.BlockSpec((ROWB, DIM), lambda i: (i, 0)),
            pl.BlockSpec((NCORE, ROWB, CC), lambda i: (0, i, 0)),
            pl.BlockSpec((DIM,), lambda i: (0,)),
            pl.BlockSpec((DIM, NFF), lambda i: (0, 0)),
            pl.BlockSpec((NFF,), lambda i: (0,)),
            pl.BlockSpec((NFF, DIM), lambda i: (0, 0)),
            pl.BlockSpec((DIM,), lambda i: (0,)),
        ],
        out_specs=pl.BlockSpec((ROWB, DIM), lambda i: (i, 0)),
        out_shape=jax.ShapeDtypeStruct((NWD, DIM), jnp.float32),
    )(word, rst, bg, w1, b1, w2, b2)


def kernel(word_repr, sent_repr, edge_index, W_src, W_dst, attn_l, attn_r,
           bias_gat, w1, b1, w2, b2):
    src = edge_index[0].astype(jnp.int32)
    dst = edge_index[1].astype(jnp.int32)
    feat2, el4, er4, mx4 = _pre(sent_repr, word_repr, W_src, W_dst,
                                attn_l, attn_r)
    el4f = el4.reshape(NG, TAB)
    er4f = er4.reshape(NG, TAB)
    alphaf = _sc_edge_softmax(el4f, er4f, mx4, src, dst)
    feat2f = feat2.reshape(NCORE * NST, CC)
    rst = _sc_aggregate(feat2f, src, dst, alphaf)
    return _ffn(word_repr, rst, bias_gat, w1, b1, w2, b2)
